# baseline (speedup 1.0000x reference)
import functools

import jax
import jax.numpy as jnp
from jax import lax
from jax.experimental import pallas as pl
from jax.experimental.pallas import tpu as pltpu
from jax.experimental.pallas import tpu_sc as plsc

Z = 128
BATCH = 16384

_info = plsc.get_sparse_core_info()
_NC, _NS, _L = _info.num_cores, _info.num_subcores, _info.num_lanes
_NW = _NC * _NS
_BPW = BATCH // _NW
_C = 128
_NCHUNK = _BPW // _C
_NYBUF = 3

_mesh = plsc.VectorSubcoreMesh(core_axis_name="c", subcore_axis_name="s")

_SPLAT_DNUMS = lax.GatherDimensionNumbers(
    offset_dims=(), collapsed_slice_dims=(0,), start_index_map=(0,))


@functools.partial(
    pl.kernel,
    mesh=_mesh,
    out_type=jax.ShapeDtypeStruct((BATCH, Z), jnp.float32),
    scratch_types=(
        [pltpu.VMEM((_BPW,), jnp.int32),
         pltpu.VMEM((_BPW,), jnp.float32)]
        + [pltpu.VMEM((_C, Z), jnp.float32)] * _NCHUNK
        + [pltpu.VMEM((_C, Z), jnp.float32)] * _NYBUF
        + [pltpu.SemaphoreType.DMA] * _NCHUNK
        + [pltpu.SemaphoreType.DMA] * _NYBUF
    ),
)
def _sc_fma_gather(z_hbm, idx_hbm, s_hbm, tab_hbm, out_hbm,
                   idx_v, s_v, r0, r1, r2, r3, y0, y1, y2,
                   gs0, gs1, gs2, gs3, zs0, zs1, zs2):
    rows = (r0, r1, r2, r3)
    ybuf = (y0, y1, y2)
    gsem = (gs0, gs1, gs2, gs3)
    zsem = (zs0, zs1, zs2)
    wid = lax.axis_index("s") * _NC + lax.axis_index("c")
    base = wid * _BPW

    zcp = [None] * _NCHUNK
    gat = [None] * _NCHUNK
    ost = [None] * _NCHUNK

    def start_z(k):
        zcp[k] = pltpu.async_copy(
            z_hbm.at[pl.ds(base + k * _C, _C)], ybuf[k % _NYBUF],
            zsem[k % _NYBUF])

    for k in range(min(_NYBUF, _NCHUNK)):
        start_z(k)
    pltpu.sync_copy(idx_hbm.at[pl.ds(base, _BPW)], idx_v)
    pltpu.sync_copy(s_hbm.at[pl.ds(base, _BPW)], s_v)
    for k in range(_NCHUNK):
        gat[k] = pltpu.async_copy(
            tab_hbm.at[idx_v.at[pl.ds(k * _C, _C)]], rows[k], gsem[k])

    def compute(k):
        rj, yj = rows[k], ybuf[k % _NYBUF]
        off = k * _C

        def body(g, carry):
            sv16 = s_v[pl.ds(off + g * _L, _L)]
            r0_ = g * _L
            for jj in range(_L):
                splat = lax.gather(
                    sv16, jnp.full((_L, 1), jj, jnp.int32),
                    _SPLAT_DNUMS, (1,),
                    mode=lax.GatherScatterMode.PROMISE_IN_BOUNDS)
                b = r0_ + jj
                for c in range(Z // _L):
                    sl = pl.ds(c * _L, _L)
                    rj[b, sl] = rj[b, sl] * splat + yj[b, sl]
            return carry

        lax.fori_loop(0, _C // _L, body, 0)

    for k in range(_NCHUNK):
        gat[k].wait()
        zcp[k].wait()
        compute(k)
        nxt = k + _NYBUF
        if nxt < _NCHUNK:
            start_z(nxt)
        ost[k] = pltpu.async_copy(
            rows[k], out_hbm.at[pl.ds(base + k * _C, _C)], gsem[k])
    for k in range(_NCHUNK):
        ost[k].wait()


def kernel(z, labels, a):
    idx = labels[0].astype(jnp.int32)
    scale = labels[1]
    table = a.T
    return _sc_fma_gather(z, idx, scale, table)


# speedup vs baseline: 1.1469x; 1.1469x over previous
import functools

import jax
import jax.numpy as jnp
from jax import lax
from jax.experimental import pallas as pl
from jax.experimental.pallas import tpu as pltpu
from jax.experimental.pallas import tpu_sc as plsc

Z = 128
BATCH = 16384

_info = plsc.get_sparse_core_info()
_NC, _NS, _L = _info.num_cores, _info.num_subcores, _info.num_lanes
_NW = _NC * _NS
_BPW = BATCH // _NW
_C = 128
_NCHUNK = _BPW // _C
_NYBUF = 3

_mesh = plsc.VectorSubcoreMesh(core_axis_name="c", subcore_axis_name="s")

_SPLAT_DNUMS = lax.GatherDimensionNumbers(
    offset_dims=(), collapsed_slice_dims=(0,), start_index_map=(0,))


@functools.partial(
    pl.kernel,
    mesh=_mesh,
    out_type=jax.ShapeDtypeStruct((BATCH, Z), jnp.float32),
    scratch_types=(
        [pltpu.VMEM((_BPW,), jnp.int32),
         pltpu.VMEM((_BPW,), jnp.float32)]
        + [pltpu.VMEM((_C, Z), jnp.float32)] * _NCHUNK
        + [pltpu.VMEM((_C, Z), jnp.float32)] * _NYBUF
        + [pltpu.SemaphoreType.DMA] * _NCHUNK
        + [pltpu.SemaphoreType.DMA] * (2 * _NYBUF)
    ),
)
def _sc_fma_gather(z_hbm, idx_hbm, s_hbm, tab_hbm, out_hbm,
                   idx_v, s_v, r0, r1, r2, r3, y0, y1, y2,
                   gs0, gs1, gs2, gs3, zs0, zs1, zs2, os0, os1, os2):
    rows = (r0, r1, r2, r3)
    ybuf = (y0, y1, y2)
    gsem = (gs0, gs1, gs2, gs3)
    zsem = (zs0, zs1, zs2)
    osem = (os0, os1, os2)
    wid = lax.axis_index("s") * _NC + lax.axis_index("c")
    base = wid * _BPW

    zcp = [None] * _NCHUNK
    gat = [None] * _NCHUNK
    ost = [None] * _NCHUNK

    def start_z(k):
        zcp[k] = pltpu.async_copy(
            z_hbm.at[pl.ds(base + k * _C, _C)], ybuf[k % _NYBUF],
            zsem[k % _NYBUF])

    for k in range(min(_NYBUF, _NCHUNK)):
        start_z(k)
    pltpu.sync_copy(idx_hbm.at[pl.ds(base, _BPW)], idx_v)
    pltpu.sync_copy(s_hbm.at[pl.ds(base, _BPW)], s_v)
    for k in range(_NCHUNK):
        gat[k] = pltpu.async_copy(
            tab_hbm.at[idx_v.at[pl.ds(k * _C, _C)]], rows[k], gsem[k])

    def compute(k):
        rj, yj = rows[k], ybuf[k % _NYBUF]
        off = k * _C

        def body(g, carry):
            sv16 = s_v[pl.ds(off + g * _L, _L)]
            r0_ = g * _L
            for jj in range(_L):
                splat = lax.gather(
                    sv16, jnp.full((_L, 1), jj, jnp.int32),
                    _SPLAT_DNUMS, (1,),
                    mode=lax.GatherScatterMode.PROMISE_IN_BOUNDS)
                b = r0_ + jj
                for c in range(Z // _L):
                    sl = pl.ds(c * _L, _L)
                    plsc.addupdate(yj.at[b, sl], rj[b, sl] * splat)
            return carry

        lax.fori_loop(0, _C // _L, body, 0)

    for k in range(_NCHUNK):
        gat[k].wait()
        zcp[k].wait()
        compute(k)
        ost[k] = pltpu.async_copy(
            ybuf[k % _NYBUF], out_hbm.at[pl.ds(base + k * _C, _C)],
            osem[k % _NYBUF])
        nxt = k + _NYBUF - 1
        if k >= 1 and nxt < _NCHUNK:
            ost[k - 1].wait()
            start_z(nxt)
    for k in range(max(0, _NCHUNK - _NYBUF + 1), _NCHUNK):
        ost[k].wait()


def kernel(z, labels, a):
    idx = labels[0].astype(jnp.int32)
    scale = labels[1]
    table = a.T
    return _sc_fma_gather(z, idx, scale, table)
